# baseline (speedup 1.0000x reference)
import functools

import jax
import jax.numpy as jnp
from jax import lax
from jax.experimental import pallas as pl
from jax.experimental.pallas import tpu as pltpu
from jax.experimental.pallas import tpu_sc as plsc

N_NODES_C = 10000
N_EDGES_C = 160000
H_C = 256

_NC = 2
_NS = 16
_NW = _NC * _NS
_LANES = 16

_CHUNK = 128
_EDGES_PAD = 163840
_NCHUNKS = _EDGES_PAD // _CHUNK
_NBUF = 2
_HW = H_C // 2

_CPW0 = 80
_CPW1 = 0
_CPW_MAX = max(_CPW0, _CPW1)
_IDX_ROWS = _NCHUNKS + _CPW_MAX


def _bf16_bits(v):
    r = v.astype(jnp.bfloat16).astype(jnp.float32)
    return jax.lax.bitcast_convert_type(r, jnp.uint32) >> 16


def _pack_pairs(v):
    n = v.shape[1] // 2
    packed = _bf16_bits(v[:, :n]) | (_bf16_bits(v[:, n:]) << 16)
    return jax.lax.bitcast_convert_type(packed, jnp.int32)


def _unpack_pairs_f32(gi):
    gu = jax.lax.bitcast_convert_type(gi, jnp.uint32)
    lo = jax.lax.bitcast_convert_type(gu << 16, jnp.float32)
    hi = jax.lax.bitcast_convert_type(gu & jnp.uint32(0xFFFF0000), jnp.float32)
    return lo, hi


def _tables_body(x_ref, wa_ref, wb_ref, b1_ref, a_ref, b_ref):
    xb = x_ref[...]
    af = jnp.dot(xb, wa_ref[...], preferred_element_type=jnp.float32) + b1_ref[...]
    bf = jnp.dot(xb, wb_ref[...], preferred_element_type=jnp.float32)
    a_ref[...] = _pack_pairs(af)
    b_ref[...] = _pack_pairs(bf)


def _make_tables(x, wa, wb, b1r):
    n, h = x.shape
    blk = 1000
    grid = n // blk
    return pl.pallas_call(
        _tables_body,
        grid=(grid,),
        in_specs=[
            pl.BlockSpec((blk, h), lambda i: (i, 0)),
            pl.BlockSpec((h, h), lambda i: (0, 0)),
            pl.BlockSpec((h, h), lambda i: (0, 0)),
            pl.BlockSpec((1, h), lambda i: (0, 0)),
        ],
        out_specs=[
            pl.BlockSpec((blk, _HW), lambda i: (i, 0)),
            pl.BlockSpec((blk, _HW), lambda i: (i, 0)),
        ],
        out_shape=[
            jax.ShapeDtypeStruct((n, _HW), jnp.int32),
            jax.ShapeDtypeStruct((n, _HW), jnp.int32),
        ],
    )(x, wa, wb, b1r)


def _sc_gather_body(
    a_hbm,
    b_hbm,
    row_hbm,
    col_hbm,
    ga_hbm,
    gb_hbm,
    ridx,
    cidx,
    idxa,
    idxb,
    bufa,
    bufb,
    sem_a,
    sem_b,
    sem_wa,
    sem_wb,
):
    cid = lax.axis_index("c")
    sid = lax.axis_index("s")
    cpw = jnp.where(cid == 0, _CPW0, _CPW1)
    cbase = jnp.where(cid == 0, sid * _CPW0, _NS * _CPW0 + sid * _CPW1)

    pltpu.sync_copy(row_hbm.at[pl.ds(cbase, _CPW_MAX)], ridx)
    pltpu.sync_copy(col_hbm.at[pl.ds(cbase, _CPW_MAX)], cidx)

    def stage_idx(k, b):
        for j in range(_CHUNK // _LANES):
            sl = pl.ds(j * _LANES, _LANES)
            idxa[b][sl] = ridx[k, sl]
            idxb[b][sl] = cidx[k, sl]

    def start_gathers(b):
        pltpu.make_async_copy(a_hbm.at[idxa[b]], bufa[b], sem_a[b]).start()
        pltpu.make_async_copy(b_hbm.at[idxb[b]], bufb[b], sem_b[b]).start()

    def wait_gathers(b):
        pltpu.make_async_copy(a_hbm.at[idxa[b]], bufa[b], sem_a[b]).wait()
        pltpu.make_async_copy(b_hbm.at[idxb[b]], bufb[b], sem_b[b]).wait()

    for b in range(_NBUF):

        @pl.when(b < cpw)
        def _():
            stage_idx(b, b)
            start_gathers(b)

    def chunk_body(c, carry):
        for b in range(_NBUF):
            k = c * _NBUF + b
            wait_gathers(b)
            off = (cbase + k) * _CHUNK
            wba = pltpu.make_async_copy(
                bufa[b], ga_hbm.at[pl.ds(off, _CHUNK)], sem_wa[b]
            )
            wbb = pltpu.make_async_copy(
                bufb[b], gb_hbm.at[pl.ds(off, _CHUNK)], sem_wb[b]
            )
            wba.start()
            wbb.start()
            wba.wait()
            wbb.wait()
            nxt = k + _NBUF

            @pl.when(nxt < cpw)
            def _():
                stage_idx(nxt, b)
                start_gathers(b)

        return carry

    lax.fori_loop(0, cpw // _NBUF, chunk_body, 0, unroll=False)


def _make_gather(a_view, b_view, row_pad, col_pad):
    mesh = plsc.VectorSubcoreMesh(
        core_axis_name="c", subcore_axis_name="s", num_cores=_NC, num_subcores=_NS
    )
    return pl.kernel(
        _sc_gather_body,
        out_type=(
            jax.ShapeDtypeStruct((_EDGES_PAD, _HW), jnp.int32),
            jax.ShapeDtypeStruct((_EDGES_PAD, _HW), jnp.int32),
        ),
        mesh=mesh,
        scratch_types=[
            pltpu.VMEM((_CPW_MAX, _CHUNK), jnp.int32),
            pltpu.VMEM((_CPW_MAX, _CHUNK), jnp.int32),
            [pltpu.VMEM((_CHUNK,), jnp.int32) for _ in range(_NBUF)],
            [pltpu.VMEM((_CHUNK,), jnp.int32) for _ in range(_NBUF)],
            [pltpu.VMEM((_CHUNK, _HW), jnp.int32) for _ in range(_NBUF)],
            [pltpu.VMEM((_CHUNK, _HW), jnp.int32) for _ in range(_NBUF)],
            [pltpu.SemaphoreType.DMA for _ in range(_NBUF)],
            [pltpu.SemaphoreType.DMA for _ in range(_NBUF)],
            [pltpu.SemaphoreType.DMA for _ in range(_NBUF)],
            [pltpu.SemaphoreType.DMA for _ in range(_NBUF)],
        ],
    )(a_view, b_view, row_pad, col_pad)


def _mlp_body(ga_ref, gb_ref, w2t_ref, b2_ref, o_ref):
    alo, ahi = _unpack_pairs_f32(ga_ref[...])
    blo, bhi = _unpack_pairs_f32(gb_ref[...])
    hlo = jnp.maximum(alo + blo, 0.0).astype(jnp.bfloat16)
    hhi = jnp.maximum(ahi + bhi, 0.0).astype(jnp.bfloat16)
    h = jnp.concatenate([hlo, hhi], axis=1)
    o_ref[...] = (
        jnp.dot(h, w2t_ref[...], preferred_element_type=jnp.float32) + b2_ref[...]
    )


def _make_mlp(ga_view, gb_view, w2t, b2r, n_edges):
    h = w2t.shape[0]
    blk = 640
    grid = n_edges // blk
    return pl.pallas_call(
        _mlp_body,
        grid=(grid,),
        in_specs=[
            pl.BlockSpec((blk, _HW), lambda i: (i, 0)),
            pl.BlockSpec((blk, _HW), lambda i: (i, 0)),
            pl.BlockSpec((h, h), lambda i: (0, 0)),
            pl.BlockSpec((1, h), lambda i: (0, 0)),
        ],
        out_specs=pl.BlockSpec((blk, h), lambda i: (i, 0)),
        out_shape=jax.ShapeDtypeStruct((n_edges, h), jnp.float32),
    )(ga_view, gb_view, w2t, b2r)


def kernel(x, edge_index, W1, b1, W2, b2):
    n, h = x.shape
    e = edge_index.shape[1]

    row = edge_index[0].astype(jnp.int32)
    col = edge_index[1].astype(jnp.int32)
    pad = _IDX_ROWS * _CHUNK - e
    row_pad = jnp.pad(row, (0, pad)).reshape(_IDX_ROWS, _CHUNK)
    col_pad = jnp.pad(col, (0, pad)).reshape(_IDX_ROWS, _CHUNK)

    w1t = W1.T
    wa = w1t[:h]
    wb = w1t[h:]
    w2t = W2.T.astype(jnp.bfloat16)
    b1r = b1.reshape(1, h)
    b2r = b2.reshape(1, h)

    a_view, b_view = _make_tables(x, wa, wb, b1r)
    ga_view, gb_view = _make_gather(a_view, b_view, row_pad, col_pad)
    out = _make_mlp(ga_view, gb_view, w2t, b2r, e)
    return out


# speedup vs baseline: 1.1080x; 1.1080x over previous
import functools

import jax
import jax.numpy as jnp
from jax import lax
from jax.experimental import pallas as pl
from jax.experimental.pallas import tpu as pltpu
from jax.experimental.pallas import tpu_sc as plsc

N_NODES_C = 10000
N_EDGES_C = 160000
H_C = 256

_NC = 2
_NS = 16
_NW = _NC * _NS
_LANES = 16

_CHUNK = 64
_EDGES_PAD = 163840
_NCHUNKS = _EDGES_PAD // _CHUNK
_NG = 4
_LOOK = 2
_HW = H_C // 2

_CPW0 = 80
_CPW1 = 80
_CPW_MAX = max(_CPW0, _CPW1)
_IDX_ROWS = _NCHUNKS + _CPW_MAX


def _bf16_bits(v):
    r = v.astype(jnp.bfloat16).astype(jnp.float32)
    return jax.lax.bitcast_convert_type(r, jnp.uint32) >> 16


def _pack_pairs(v):
    n = v.shape[1] // 2
    packed = _bf16_bits(v[:, :n]) | (_bf16_bits(v[:, n:]) << 16)
    return jax.lax.bitcast_convert_type(packed, jnp.int32)


def _unpack_pairs_f32(gi):
    gu = jax.lax.bitcast_convert_type(gi, jnp.uint32)
    lo = jax.lax.bitcast_convert_type(gu << 16, jnp.float32)
    hi = jax.lax.bitcast_convert_type(gu & jnp.uint32(0xFFFF0000), jnp.float32)
    return lo, hi


def _tables_body(x_ref, wa_ref, wb_ref, b1_ref, a_ref, b_ref):
    xb = x_ref[...]
    af = jnp.dot(xb, wa_ref[...], preferred_element_type=jnp.float32) + b1_ref[...]
    bf = jnp.dot(xb, wb_ref[...], preferred_element_type=jnp.float32)
    a_ref[...] = _pack_pairs(af)
    b_ref[...] = _pack_pairs(bf)


def _make_tables(x, wa, wb, b1r):
    n, h = x.shape
    blk = 1000
    grid = n // blk
    return pl.pallas_call(
        _tables_body,
        grid=(grid,),
        in_specs=[
            pl.BlockSpec((blk, h), lambda i: (i, 0)),
            pl.BlockSpec((h, h), lambda i: (0, 0)),
            pl.BlockSpec((h, h), lambda i: (0, 0)),
            pl.BlockSpec((1, h), lambda i: (0, 0)),
        ],
        out_specs=[
            pl.BlockSpec((blk, _HW), lambda i: (i, 0)),
            pl.BlockSpec((blk, _HW), lambda i: (i, 0)),
        ],
        out_shape=[
            jax.ShapeDtypeStruct((n, _HW), jnp.int32),
            jax.ShapeDtypeStruct((n, _HW), jnp.int32),
        ],
    )(x, wa, wb, b1r)


def _sc_gather_body(
    a_hbm,
    b_hbm,
    row_hbm,
    col_hbm,
    ga_hbm,
    gb_hbm,
    ridx,
    cidx,
    idxa,
    idxb,
    bufa,
    bufb,
    sem_a,
    sem_b,
    sem_wa,
    sem_wb,
):
    cid = lax.axis_index("c")
    sid = lax.axis_index("s")
    cpw = jnp.where(cid == 0, _CPW0, _CPW1)
    cbase = jnp.where(cid == 0, sid * _CPW0, _NS * _CPW0 + sid * _CPW1)

    pltpu.sync_copy(row_hbm.at[pl.ds(cbase, _CPW_MAX)], ridx)
    pltpu.sync_copy(col_hbm.at[pl.ds(cbase, _CPW_MAX)], cidx)

    def stage_idx(k, g):
        for j in range(_CHUNK // _LANES):
            sl = pl.ds(j * _LANES, _LANES)
            idxa[g][sl] = ridx[k, sl]
            idxb[g][sl] = cidx[k, sl]

    def start_gathers(g):
        pltpu.make_async_copy(a_hbm.at[idxa[g]], bufa[g], sem_a[g]).start()
        pltpu.make_async_copy(b_hbm.at[idxb[g]], bufb[g], sem_b[g]).start()

    def wait_gathers(g):
        pltpu.make_async_copy(a_hbm.at[idxa[g]], bufa[g], sem_a[g]).wait()
        pltpu.make_async_copy(b_hbm.at[idxb[g]], bufb[g], sem_b[g]).wait()

    def wait_wbs(g):
        pltpu.make_async_copy(bufa[g], ga_hbm.at[pl.ds(0, _CHUNK)], sem_wa[g]).wait()
        pltpu.make_async_copy(bufb[g], gb_hbm.at[pl.ds(0, _CHUNK)], sem_wb[g]).wait()

    for g in range(_LOOK):

        @pl.when(g < cpw)
        def _():
            stage_idx(g, g)
            start_gathers(g)

    def chunk_body(t, carry):
        for b in range(_NG):
            k = t * _NG + b
            wait_gathers(b)
            off = (cbase + k) * _CHUNK
            pltpu.make_async_copy(
                bufa[b], ga_hbm.at[pl.ds(off, _CHUNK)], sem_wa[b]
            ).start()
            pltpu.make_async_copy(
                bufb[b], gb_hbm.at[pl.ds(off, _CHUNK)], sem_wb[b]
            ).start()
            nxt = k + _LOOK
            s = (b + _LOOK) % _NG

            @pl.when(nxt < cpw)
            def _():
                @pl.when(nxt - _NG >= 0)
                def _():
                    wait_wbs(s)

                stage_idx(nxt, s)
                start_gathers(s)

        return carry

    lax.fori_loop(0, cpw // _NG, chunk_body, 0, unroll=False)
    for g in range(_NG):

        @pl.when(cpw > 0)
        def _():
            wait_wbs(g)


def _make_gather(a_view, b_view, row_pad, col_pad):
    mesh = plsc.VectorSubcoreMesh(
        core_axis_name="c", subcore_axis_name="s", num_cores=_NC, num_subcores=_NS
    )
    return pl.kernel(
        _sc_gather_body,
        out_type=(
            jax.ShapeDtypeStruct((_EDGES_PAD, _HW), jnp.int32),
            jax.ShapeDtypeStruct((_EDGES_PAD, _HW), jnp.int32),
        ),
        mesh=mesh,
        scratch_types=[
            pltpu.VMEM((_CPW_MAX, _CHUNK), jnp.int32),
            pltpu.VMEM((_CPW_MAX, _CHUNK), jnp.int32),
            [pltpu.VMEM((_CHUNK,), jnp.int32) for _ in range(_NG)],
            [pltpu.VMEM((_CHUNK,), jnp.int32) for _ in range(_NG)],
            [pltpu.VMEM((_CHUNK, _HW), jnp.int32) for _ in range(_NG)],
            [pltpu.VMEM((_CHUNK, _HW), jnp.int32) for _ in range(_NG)],
            [pltpu.SemaphoreType.DMA for _ in range(_NG)],
            [pltpu.SemaphoreType.DMA for _ in range(_NG)],
            [pltpu.SemaphoreType.DMA for _ in range(_NG)],
            [pltpu.SemaphoreType.DMA for _ in range(_NG)],
        ],
    )(a_view, b_view, row_pad, col_pad)


def _mlp_body(ga_ref, gb_ref, w2t_ref, b2_ref, o_ref):
    alo, ahi = _unpack_pairs_f32(ga_ref[...])
    blo, bhi = _unpack_pairs_f32(gb_ref[...])
    hlo = jnp.maximum(alo + blo, 0.0).astype(jnp.bfloat16)
    hhi = jnp.maximum(ahi + bhi, 0.0).astype(jnp.bfloat16)
    h = jnp.concatenate([hlo, hhi], axis=1)
    o_ref[...] = (
        jnp.dot(h, w2t_ref[...], preferred_element_type=jnp.float32) + b2_ref[...]
    )


def _make_mlp(ga_view, gb_view, w2t, b2r, n_edges):
    h = w2t.shape[0]
    blk = 640
    grid = n_edges // blk
    return pl.pallas_call(
        _mlp_body,
        grid=(grid,),
        in_specs=[
            pl.BlockSpec((blk, _HW), lambda i: (i, 0)),
            pl.BlockSpec((blk, _HW), lambda i: (i, 0)),
            pl.BlockSpec((h, h), lambda i: (0, 0)),
            pl.BlockSpec((1, h), lambda i: (0, 0)),
        ],
        out_specs=pl.BlockSpec((blk, h), lambda i: (i, 0)),
        out_shape=jax.ShapeDtypeStruct((n_edges, h), jnp.float32),
    )(ga_view, gb_view, w2t, b2r)


def kernel(x, edge_index, W1, b1, W2, b2):
    n, h = x.shape
    e = edge_index.shape[1]

    row = edge_index[0].astype(jnp.int32)
    col = edge_index[1].astype(jnp.int32)
    pad = _IDX_ROWS * _CHUNK - e
    row_pad = jnp.pad(row, (0, pad)).reshape(_IDX_ROWS, _CHUNK)
    col_pad = jnp.pad(col, (0, pad)).reshape(_IDX_ROWS, _CHUNK)

    w1t = W1.T
    wa = w1t[:h]
    wb = w1t[h:]
    w2t = W2.T.astype(jnp.bfloat16)
    b1r = b1.reshape(1, h)
    b2r = b2.reshape(1, h)

    a_view, b_view = _make_tables(x, wa, wb, b1r)
    ga_view, gb_view = _make_gather(a_view, b_view, row_pad, col_pad)
    out = _make_mlp(ga_view, gb_view, w2t, b2r, e)
    return out


# speedup vs baseline: 1.2277x; 1.1080x over previous
import functools

import jax
import jax.numpy as jnp
from jax import lax
from jax.experimental import pallas as pl
from jax.experimental.pallas import tpu as pltpu
from jax.experimental.pallas import tpu_sc as plsc

N_NODES_C = 10000
N_EDGES_C = 160000
H_C = 256

_NC = 2
_NS = 16
_NW = _NC * _NS
_LANES = 16

_CHUNK = 64
_EDGES_PAD = 163840
_NCHUNKS = _EDGES_PAD // _CHUNK
_NG = 4
_LOOK = 2
_HW = H_C // 2

_CPW0 = 80
_CPW1 = 80
_CPW_MAX = max(_CPW0, _CPW1)
_IDX_ROWS = _NCHUNKS + _CPW_MAX


def _bf16_bits(v):
    r = v.astype(jnp.bfloat16).astype(jnp.float32)
    return jax.lax.bitcast_convert_type(r, jnp.uint32) >> 16


def _pack_pairs(v):
    n = v.shape[1] // 2
    packed = _bf16_bits(v[:, :n]) | (_bf16_bits(v[:, n:]) << 16)
    return jax.lax.bitcast_convert_type(packed, jnp.int32)


def _unpack_pairs_f32(gi):
    gu = jax.lax.bitcast_convert_type(gi, jnp.uint32)
    lo = jax.lax.bitcast_convert_type(gu << 16, jnp.float32)
    hi = jax.lax.bitcast_convert_type(gu & jnp.uint32(0xFFFF0000), jnp.float32)
    return lo, hi


def _tables_body(x_ref, wa_ref, wb_ref, b1_ref, a_ref, b_ref):
    xb = x_ref[...]
    af = jnp.dot(xb, wa_ref[...], preferred_element_type=jnp.float32) + b1_ref[...]
    bf = jnp.dot(xb, wb_ref[...], preferred_element_type=jnp.float32)
    a_ref[...] = _pack_pairs(af)
    b_ref[...] = _pack_pairs(bf)


def _make_tables(x, wa, wb, b1r):
    n, h = x.shape
    blk = 1000
    grid = n // blk
    return pl.pallas_call(
        _tables_body,
        grid=(grid,),
        in_specs=[
            pl.BlockSpec((blk, h), lambda i: (i, 0)),
            pl.BlockSpec((h, h), lambda i: (0, 0)),
            pl.BlockSpec((h, h), lambda i: (0, 0)),
            pl.BlockSpec((1, h), lambda i: (0, 0)),
        ],
        out_specs=[
            pl.BlockSpec((blk, _HW), lambda i: (i, 0)),
            pl.BlockSpec((blk, _HW), lambda i: (i, 0)),
        ],
        out_shape=[
            jax.ShapeDtypeStruct((n, _HW), jnp.int32),
            jax.ShapeDtypeStruct((n, _HW), jnp.int32),
        ],
    )(x, wa, wb, b1r)


def _sc_gather_body(
    a_hbm,
    b_hbm,
    row_hbm,
    col_hbm,
    ga_hbm,
    gb_hbm,
    ridx,
    cidx,
    idxa,
    idxb,
    bufa,
    bufb,
    sem_a,
    sem_b,
    sem_wa,
    sem_wb,
):
    cid = lax.axis_index("c")
    sid = lax.axis_index("s")
    cpw = jnp.where(cid == 0, _CPW0, _CPW1)
    cbase = jnp.where(cid == 0, sid * _CPW0, _NS * _CPW0 + sid * _CPW1)

    pltpu.sync_copy(row_hbm.at[pl.ds(cbase, _CPW_MAX)], ridx)
    pltpu.sync_copy(col_hbm.at[pl.ds(cbase, _CPW_MAX)], cidx)

    def stage_idx(k, g):
        for j in range(_CHUNK // _LANES):
            sl = pl.ds(j * _LANES, _LANES)
            idxa[g][sl] = ridx[k, sl]
            idxb[g][sl] = cidx[k, sl]

    def start_gathers(g):
        pltpu.make_async_copy(a_hbm.at[idxa[g]], bufa[g], sem_a[g]).start()
        pltpu.make_async_copy(b_hbm.at[idxb[g]], bufb[g], sem_b[g]).start()

    def wait_gathers(g):
        pltpu.make_async_copy(a_hbm.at[idxa[g]], bufa[g], sem_a[g]).wait()
        pltpu.make_async_copy(b_hbm.at[idxb[g]], bufb[g], sem_b[g]).wait()

    def wait_wbs(g):
        pltpu.make_async_copy(bufa[g], ga_hbm.at[pl.ds(0, _CHUNK)], sem_wa[g]).wait()
        pltpu.make_async_copy(bufb[g], gb_hbm.at[pl.ds(0, _CHUNK)], sem_wb[g]).wait()

    for g in range(_LOOK):

        @pl.when(g < cpw)
        def _():
            stage_idx(g, g)
            start_gathers(g)

    def chunk_body(t, carry):
        for b in range(_NG):
            k = t * _NG + b
            wait_gathers(b)
            off = (cbase + k) * _CHUNK
            pltpu.make_async_copy(
                bufa[b], ga_hbm.at[pl.ds(off, _CHUNK)], sem_wa[b]
            ).start()
            pltpu.make_async_copy(
                bufb[b], gb_hbm.at[pl.ds(off, _CHUNK)], sem_wb[b]
            ).start()
            nxt = k + _LOOK
            s = (b + _LOOK) % _NG

            @pl.when(nxt < cpw)
            def _():
                @pl.when(nxt - _NG >= 0)
                def _():
                    wait_wbs(s)

                stage_idx(nxt, s)
                start_gathers(s)

        return carry

    lax.fori_loop(0, cpw // _NG, chunk_body, 0, unroll=False)
    for g in range(_NG):

        @pl.when(cpw > 0)
        def _():
            wait_wbs(g)


def _make_gather(a_view, b_view, row_pad, col_pad):
    mesh = plsc.VectorSubcoreMesh(
        core_axis_name="c", subcore_axis_name="s", num_cores=_NC, num_subcores=_NS
    )
    return pl.kernel(
        _sc_gather_body,
        out_type=(
            jax.ShapeDtypeStruct((_EDGES_PAD, _HW), jnp.int32),
            jax.ShapeDtypeStruct((_EDGES_PAD, _HW), jnp.int32),
        ),
        mesh=mesh,
        scratch_types=[
            pltpu.VMEM((_CPW_MAX, _CHUNK), jnp.int32),
            pltpu.VMEM((_CPW_MAX, _CHUNK), jnp.int32),
            [pltpu.VMEM((_CHUNK,), jnp.int32) for _ in range(_NG)],
            [pltpu.VMEM((_CHUNK,), jnp.int32) for _ in range(_NG)],
            [pltpu.VMEM((_CHUNK, _HW), jnp.int32) for _ in range(_NG)],
            [pltpu.VMEM((_CHUNK, _HW), jnp.int32) for _ in range(_NG)],
            [pltpu.SemaphoreType.DMA for _ in range(_NG)],
            [pltpu.SemaphoreType.DMA for _ in range(_NG)],
            [pltpu.SemaphoreType.DMA for _ in range(_NG)],
            [pltpu.SemaphoreType.DMA for _ in range(_NG)],
        ],
    )(a_view, b_view, row_pad, col_pad)


def _mlp_body(ga_ref, gb_ref, w2t_ref, b2_ref, o_ref):
    alo, ahi = _unpack_pairs_f32(ga_ref[...])
    blo, bhi = _unpack_pairs_f32(gb_ref[...])
    hlo = jnp.maximum(alo + blo, 0.0).astype(jnp.bfloat16)
    hhi = jnp.maximum(ahi + bhi, 0.0).astype(jnp.bfloat16)
    o_ref[...] = (
        jnp.dot(hlo, w2t_ref[: hlo.shape[1]], preferred_element_type=jnp.float32)
        + jnp.dot(hhi, w2t_ref[hlo.shape[1] :], preferred_element_type=jnp.float32)
        + b2_ref[...]
    )


def _make_mlp(ga_view, gb_view, w2t_perm, b2r, n_edges):
    h = w2t_perm.shape[0]
    blk = 1280
    grid = n_edges // blk
    return pl.pallas_call(
        _mlp_body,
        grid=(grid,),
        in_specs=[
            pl.BlockSpec((blk, _HW), lambda i: (i, 0)),
            pl.BlockSpec((blk, _HW), lambda i: (i, 0)),
            pl.BlockSpec((h, h), lambda i: (0, 0)),
            pl.BlockSpec((1, h), lambda i: (0, 0)),
        ],
        out_specs=pl.BlockSpec((blk, h), lambda i: (i, 0)),
        out_shape=jax.ShapeDtypeStruct((n_edges, h), jnp.float32),
    )(ga_view, gb_view, w2t_perm, b2r)


def kernel(x, edge_index, W1, b1, W2, b2):
    n, h = x.shape
    e = edge_index.shape[1]

    row = edge_index[0].astype(jnp.int32)
    col = edge_index[1].astype(jnp.int32)
    pad = _IDX_ROWS * _CHUNK - e
    row_pad = jnp.pad(row, (0, pad)).reshape(_IDX_ROWS, _CHUNK)
    col_pad = jnp.pad(col, (0, pad)).reshape(_IDX_ROWS, _CHUNK)

    w1t = W1.T
    wa = w1t[:h]
    wb = w1t[h:]
    w2t_perm = W2.T.astype(jnp.bfloat16)
    b1r = b1.reshape(1, h)
    b2r = b2.reshape(1, h)

    a_view, b_view = _make_tables(x, wa, wb, b1r)
    ga_view, gb_view = _make_gather(a_view, b_view, row_pad, col_pad)
    out = _make_mlp(ga_view, gb_view, w2t_perm, b2r, e)
    return out


# speedup vs baseline: 1.2527x; 1.0204x over previous
import functools

import jax
import jax.numpy as jnp
from jax import lax
from jax.experimental import pallas as pl
from jax.experimental.pallas import tpu as pltpu
from jax.experimental.pallas import tpu_sc as plsc

N_NODES_C = 10000
N_EDGES_C = 160000
H_C = 256

_NC = 2
_NS = 16
_NW = _NC * _NS
_LANES = 16

_CHUNK = 64
_EDGES_PAD = 163840
_NCHUNKS = _EDGES_PAD // _CHUNK
_NG = 4
_LOOK = 2
_HW = H_C // 2

_CPW0 = 80
_CPW1 = 80
_CPW_MAX = max(_CPW0, _CPW1)
_IDX_ROWS = _NCHUNKS + _CPW_MAX


def _bf16_bits(v):
    r = v.astype(jnp.bfloat16).astype(jnp.float32)
    return jax.lax.bitcast_convert_type(r, jnp.uint32) >> 16


def _pack_pairs(v):
    n = v.shape[1] // 2
    packed = _bf16_bits(v[:, :n]) | (_bf16_bits(v[:, n:]) << 16)
    return jax.lax.bitcast_convert_type(packed, jnp.int32)


def _unpack_pairs_f32(gi):
    gu = jax.lax.bitcast_convert_type(gi, jnp.uint32)
    lo = jax.lax.bitcast_convert_type(gu << 16, jnp.float32)
    hi = jax.lax.bitcast_convert_type(gu & jnp.uint32(0xFFFF0000), jnp.float32)
    return lo, hi


def _tables_body(x_ref, wa_ref, wb_ref, b1_ref, a_ref, b_ref):
    xb = x_ref[...]
    af = jnp.dot(xb, wa_ref[...], preferred_element_type=jnp.float32) + b1_ref[...]
    bf = jnp.dot(xb, wb_ref[...], preferred_element_type=jnp.float32)
    a_ref[...] = _pack_pairs(af)
    b_ref[...] = _pack_pairs(bf)


def _make_tables(x, wa, wb, b1r):
    n, h = x.shape
    blk = 1000
    grid = n // blk
    return pl.pallas_call(
        _tables_body,
        grid=(grid,),
        in_specs=[
            pl.BlockSpec((blk, h), lambda i: (i, 0)),
            pl.BlockSpec((h, h), lambda i: (0, 0)),
            pl.BlockSpec((h, h), lambda i: (0, 0)),
            pl.BlockSpec((1, h), lambda i: (0, 0)),
        ],
        out_specs=[
            pl.BlockSpec((blk, _HW), lambda i: (i, 0)),
            pl.BlockSpec((blk, _HW), lambda i: (i, 0)),
        ],
        out_shape=[
            jax.ShapeDtypeStruct((n, _HW), jnp.int32),
            jax.ShapeDtypeStruct((n, _HW), jnp.int32),
        ],
    )(x, wa, wb, b1r)


def _sc_gather_body(
    a_hbm,
    b_hbm,
    row_hbm,
    col_hbm,
    ga_hbm,
    gb_hbm,
    ridx,
    cidx,
    idxa,
    idxb,
    bufa,
    bufb,
    sem_a,
    sem_b,
    sem_wa,
    sem_wb,
):
    cid = lax.axis_index("c")
    sid = lax.axis_index("s")
    cpw = jnp.where(cid == 0, _CPW0, _CPW1)
    cbase = jnp.where(cid == 0, sid * _CPW0, _NS * _CPW0 + sid * _CPW1)

    pltpu.sync_copy(row_hbm.at[pl.ds(cbase, _CPW_MAX)], ridx)
    pltpu.sync_copy(col_hbm.at[pl.ds(cbase, _CPW_MAX)], cidx)

    def stage_idx(k, g):
        for j in range(_CHUNK // _LANES):
            sl = pl.ds(j * _LANES, _LANES)
            idxa[g][sl] = ridx[k, sl]
            idxb[g][sl] = cidx[k, sl]

    def start_gathers(g):
        pltpu.make_async_copy(a_hbm.at[idxa[g]], bufa[g], sem_a[g]).start()
        pltpu.make_async_copy(b_hbm.at[idxb[g]], bufb[g], sem_b[g]).start()

    def wait_gathers(g):
        pltpu.make_async_copy(a_hbm.at[idxa[g]], bufa[g], sem_a[g]).wait()
        pltpu.make_async_copy(b_hbm.at[idxb[g]], bufb[g], sem_b[g]).wait()

    def wait_wbs(g):
        pltpu.make_async_copy(bufa[g], ga_hbm.at[pl.ds(0, _CHUNK)], sem_wa[g]).wait()
        pltpu.make_async_copy(bufb[g], gb_hbm.at[pl.ds(0, _CHUNK)], sem_wb[g]).wait()

    for g in range(_LOOK):

        @pl.when(g < cpw)
        def _():
            stage_idx(g, g)
            start_gathers(g)

    def chunk_body(t, carry):
        for b in range(_NG):
            k = t * _NG + b
            wait_gathers(b)
            off = (cbase + k) * _CHUNK
            pltpu.make_async_copy(
                bufa[b], ga_hbm.at[pl.ds(off, _CHUNK)], sem_wa[b]
            ).start()
            pltpu.make_async_copy(
                bufb[b], gb_hbm.at[pl.ds(off, _CHUNK)], sem_wb[b]
            ).start()
            nxt = k + _LOOK
            s = (b + _LOOK) % _NG

            @pl.when(nxt < cpw)
            def _():
                @pl.when(nxt - _NG >= 0)
                def _():
                    wait_wbs(s)

                stage_idx(nxt, s)
                start_gathers(s)

        return carry

    lax.fori_loop(0, cpw // _NG, chunk_body, 0, unroll=False)
    for g in range(_NG):

        @pl.when(cpw > 0)
        def _():
            wait_wbs(g)


def _make_gather(a_view, b_view, row_pad, col_pad):
    mesh = plsc.VectorSubcoreMesh(
        core_axis_name="c", subcore_axis_name="s", num_cores=_NC, num_subcores=_NS
    )
    return pl.kernel(
        _sc_gather_body,
        out_type=(
            jax.ShapeDtypeStruct((_EDGES_PAD, _HW), jnp.int32),
            jax.ShapeDtypeStruct((_EDGES_PAD, _HW), jnp.int32),
        ),
        mesh=mesh,
        scratch_types=[
            pltpu.VMEM((_CPW_MAX, _CHUNK), jnp.int32),
            pltpu.VMEM((_CPW_MAX, _CHUNK), jnp.int32),
            [pltpu.VMEM((_CHUNK,), jnp.int32) for _ in range(_NG)],
            [pltpu.VMEM((_CHUNK,), jnp.int32) for _ in range(_NG)],
            [pltpu.VMEM((_CHUNK, _HW), jnp.int32) for _ in range(_NG)],
            [pltpu.VMEM((_CHUNK, _HW), jnp.int32) for _ in range(_NG)],
            [pltpu.SemaphoreType.DMA for _ in range(_NG)],
            [pltpu.SemaphoreType.DMA for _ in range(_NG)],
            [pltpu.SemaphoreType.DMA for _ in range(_NG)],
            [pltpu.SemaphoreType.DMA for _ in range(_NG)],
        ],
    )(a_view, b_view, row_pad, col_pad)


def _mlp_body(ga_ref, gb_ref, w2t_ref, b2_ref, o_ref):
    alo, ahi = _unpack_pairs_f32(ga_ref[...])
    blo, bhi = _unpack_pairs_f32(gb_ref[...])
    hlo = jnp.maximum(alo + blo, 0.0).astype(jnp.bfloat16)
    hhi = jnp.maximum(ahi + bhi, 0.0).astype(jnp.bfloat16)
    h = jnp.concatenate([hlo, hhi], axis=1)
    o_ref[...] = (
        jnp.dot(h, w2t_ref[...], preferred_element_type=jnp.float32) + b2_ref[...]
    )


def _make_mlp(ga_view, gb_view, w2t_perm, b2r, n_edges):
    h = w2t_perm.shape[0]
    blk = 1280
    grid = n_edges // blk
    return pl.pallas_call(
        _mlp_body,
        grid=(grid,),
        in_specs=[
            pl.BlockSpec((blk, _HW), lambda i: (i, 0)),
            pl.BlockSpec((blk, _HW), lambda i: (i, 0)),
            pl.BlockSpec((h, h), lambda i: (0, 0)),
            pl.BlockSpec((1, h), lambda i: (0, 0)),
        ],
        out_specs=pl.BlockSpec((blk, h), lambda i: (i, 0)),
        out_shape=jax.ShapeDtypeStruct((n_edges, h), jnp.float32),
    )(ga_view, gb_view, w2t_perm, b2r)


def kernel(x, edge_index, W1, b1, W2, b2):
    n, h = x.shape
    e = edge_index.shape[1]

    row = edge_index[0].astype(jnp.int32)
    col = edge_index[1].astype(jnp.int32)
    pad = _IDX_ROWS * _CHUNK - e
    row_pad = jnp.pad(row, (0, pad)).reshape(_IDX_ROWS, _CHUNK)
    col_pad = jnp.pad(col, (0, pad)).reshape(_IDX_ROWS, _CHUNK)

    w1t = W1.T
    wa = w1t[:h]
    wb = w1t[h:]
    w2t_perm = W2.T.astype(jnp.bfloat16)
    b1r = b1.reshape(1, h)
    b2r = b2.reshape(1, h)

    a_view, b_view = _make_tables(x, wa, wb, b1r)
    ga_view, gb_view = _make_gather(a_view, b_view, row_pad, col_pad)
    out = _make_mlp(ga_view, gb_view, w2t_perm, b2r, e)
    return out


# speedup vs baseline: 1.3013x; 1.0388x over previous
import functools

import jax
import jax.numpy as jnp
from jax import lax
from jax.experimental import pallas as pl
from jax.experimental.pallas import tpu as pltpu
from jax.experimental.pallas import tpu_sc as plsc

N_NODES_C = 10000
N_EDGES_C = 160000
H_C = 256

_NC = 2
_NS = 16
_NW = _NC * _NS
_LANES = 16

_CHUNK = 64
_EDGES_PAD = 163840
_NCHUNKS = _EDGES_PAD // _CHUNK
_NG = 4
_LOOK = 2
_HW = H_C // 2

_CPW0 = 80
_CPW1 = 80
_CPW_MAX = max(_CPW0, _CPW1)
_IDX_ROWS = _NCHUNKS + _CPW_MAX


def _bf16_bits(v):
    r = v.astype(jnp.bfloat16).astype(jnp.float32)
    return jax.lax.bitcast_convert_type(r, jnp.uint32) >> 16


def _pack_pairs(v):
    n = v.shape[1] // 2
    packed = _bf16_bits(v[:, :n]) | (_bf16_bits(v[:, n:]) << 16)
    return jax.lax.bitcast_convert_type(packed, jnp.int32)


def _unpack_pairs_f32(gi):
    gu = jax.lax.bitcast_convert_type(gi, jnp.uint32)
    lo = jax.lax.bitcast_convert_type(gu << 16, jnp.float32)
    hi = jax.lax.bitcast_convert_type(gu & jnp.uint32(0xFFFF0000), jnp.float32)
    return lo, hi


def _tables_body(x_ref, wa_ref, wb_ref, b1_ref, a_ref, b_ref):
    xb = x_ref[...]
    af = jnp.dot(xb, wa_ref[...], preferred_element_type=jnp.float32) + b1_ref[...]
    bf = jnp.dot(xb, wb_ref[...], preferred_element_type=jnp.float32)
    a_ref[...] = _pack_pairs(af)
    b_ref[...] = _pack_pairs(bf)


def _make_tables(x, wa, wb, b1r):
    n, h = x.shape
    blk = 1000
    grid = n // blk
    return pl.pallas_call(
        _tables_body,
        grid=(grid,),
        in_specs=[
            pl.BlockSpec((blk, h), lambda i: (i, 0)),
            pl.BlockSpec((h, h), lambda i: (0, 0)),
            pl.BlockSpec((h, h), lambda i: (0, 0)),
            pl.BlockSpec((1, h), lambda i: (0, 0)),
        ],
        out_specs=[
            pl.BlockSpec((blk, _HW), lambda i: (i, 0)),
            pl.BlockSpec((blk, _HW), lambda i: (i, 0)),
        ],
        out_shape=[
            jax.ShapeDtypeStruct((n, _HW), jnp.int32),
            jax.ShapeDtypeStruct((n, _HW), jnp.int32),
        ],
    )(x, wa, wb, b1r)


def _sc_gather_body(
    a_hbm,
    b_hbm,
    row_hbm,
    col_hbm,
    ga_hbm,
    gb_hbm,
    ridx,
    cidx,
    idxa,
    idxb,
    bufa,
    bufb,
    sem_a,
    sem_b,
    sem_wa,
    sem_wb,
):
    cid = lax.axis_index("c")
    sid = lax.axis_index("s")
    cpw = jnp.where(cid == 0, _CPW0, _CPW1)
    cbase = jnp.where(cid == 0, sid * _CPW0, _NS * _CPW0 + sid * _CPW1)

    pltpu.sync_copy(row_hbm.at[pl.ds(cbase, _CPW_MAX)], ridx)
    pltpu.sync_copy(col_hbm.at[pl.ds(cbase, _CPW_MAX)], cidx)

    def stage_idx(k, g):
        for j in range(_CHUNK // _LANES):
            sl = pl.ds(j * _LANES, _LANES)
            idxa[g][sl] = ridx[k, sl]
            idxb[g][sl] = cidx[k, sl]

    def start_gathers(g):
        pltpu.make_async_copy(a_hbm.at[idxa[g]], bufa[g], sem_a[g]).start()
        pltpu.make_async_copy(b_hbm.at[idxb[g]], bufb[g], sem_b[g]).start()

    def wait_gathers(g):
        pltpu.make_async_copy(a_hbm.at[idxa[g]], bufa[g], sem_a[g]).wait()
        pltpu.make_async_copy(b_hbm.at[idxb[g]], bufb[g], sem_b[g]).wait()

    def wait_wbs(g):
        pltpu.make_async_copy(bufa[g], ga_hbm.at[pl.ds(0, _CHUNK)], sem_wa[g]).wait()
        pltpu.make_async_copy(bufb[g], gb_hbm.at[pl.ds(0, _CHUNK)], sem_wb[g]).wait()

    for g in range(_LOOK):

        @pl.when(g < cpw)
        def _():
            stage_idx(g, g)
            start_gathers(g)

    def chunk_body(t, carry):
        for b in range(_NG):
            k = t * _NG + b
            wait_gathers(b)
            off = (cbase + k) * _CHUNK
            pltpu.make_async_copy(
                bufa[b], ga_hbm.at[pl.ds(off, _CHUNK)], sem_wa[b]
            ).start()
            pltpu.make_async_copy(
                bufb[b], gb_hbm.at[pl.ds(off, _CHUNK)], sem_wb[b]
            ).start()
            nxt = k + _LOOK
            s = (b + _LOOK) % _NG

            @pl.when(nxt < cpw)
            def _():
                @pl.when(nxt - _NG >= 0)
                def _():
                    wait_wbs(s)

                stage_idx(nxt, s)
                start_gathers(s)

        return carry

    lax.fori_loop(0, cpw // _NG, chunk_body, 0, unroll=False)
    for g in range(_NG):

        @pl.when(cpw > 0)
        def _():
            wait_wbs(g)


def _make_gather(a_view, b_view, row_pad, col_pad):
    mesh = plsc.VectorSubcoreMesh(
        core_axis_name="c", subcore_axis_name="s", num_cores=_NC, num_subcores=_NS
    )
    return pl.kernel(
        _sc_gather_body,
        out_type=(
            jax.ShapeDtypeStruct((_EDGES_PAD, _HW), jnp.int32),
            jax.ShapeDtypeStruct((_EDGES_PAD, _HW), jnp.int32),
        ),
        mesh=mesh,
        scratch_types=[
            pltpu.VMEM((_CPW_MAX, _CHUNK), jnp.int32),
            pltpu.VMEM((_CPW_MAX, _CHUNK), jnp.int32),
            [pltpu.VMEM((_CHUNK,), jnp.int32) for _ in range(_NG)],
            [pltpu.VMEM((_CHUNK,), jnp.int32) for _ in range(_NG)],
            [pltpu.VMEM((_CHUNK, _HW), jnp.int32) for _ in range(_NG)],
            [pltpu.VMEM((_CHUNK, _HW), jnp.int32) for _ in range(_NG)],
            [pltpu.SemaphoreType.DMA for _ in range(_NG)],
            [pltpu.SemaphoreType.DMA for _ in range(_NG)],
            [pltpu.SemaphoreType.DMA for _ in range(_NG)],
            [pltpu.SemaphoreType.DMA for _ in range(_NG)],
        ],
    )(a_view, b_view, row_pad, col_pad)


def _mlp_body(ga_ref, gb_ref, w2t_ref, b2_ref, o_ref):
    alo, ahi = _unpack_pairs_f32(ga_ref[...])
    blo, bhi = _unpack_pairs_f32(gb_ref[...])
    hlo = jnp.maximum(alo + blo, 0.0).astype(jnp.bfloat16)
    hhi = jnp.maximum(ahi + bhi, 0.0).astype(jnp.bfloat16)
    h = jnp.concatenate([hlo, hhi], axis=1)
    o_ref[...] = (
        jnp.dot(h, w2t_ref[...], preferred_element_type=jnp.float32) + b2_ref[...]
    )


def _make_mlp(ga_view, gb_view, w2t_perm, b2r, n_edges):
    h = w2t_perm.shape[0]
    blk = 1600
    grid = n_edges // blk
    return pl.pallas_call(
        _mlp_body,
        grid=(grid,),
        in_specs=[
            pl.BlockSpec((blk, _HW), lambda i: (i, 0)),
            pl.BlockSpec((blk, _HW), lambda i: (i, 0)),
            pl.BlockSpec((h, h), lambda i: (0, 0)),
            pl.BlockSpec((1, h), lambda i: (0, 0)),
        ],
        out_specs=pl.BlockSpec((blk, h), lambda i: (i, 0)),
        out_shape=jax.ShapeDtypeStruct((n_edges, h), jnp.float32),
    )(ga_view, gb_view, w2t_perm, b2r)


def kernel(x, edge_index, W1, b1, W2, b2):
    n, h = x.shape
    e = edge_index.shape[1]

    row = edge_index[0].astype(jnp.int32)
    col = edge_index[1].astype(jnp.int32)
    pad = _IDX_ROWS * _CHUNK - e
    row_pad = jnp.pad(row, (0, pad)).reshape(_IDX_ROWS, _CHUNK)
    col_pad = jnp.pad(col, (0, pad)).reshape(_IDX_ROWS, _CHUNK)

    w1t = W1.T
    wa = w1t[:h]
    wb = w1t[h:]
    w2t_perm = W2.T.astype(jnp.bfloat16)
    b1r = b1.reshape(1, h)
    b2r = b2.reshape(1, h)

    a_view, b_view = _make_tables(x, wa, wb, b1r)
    ga_view, gb_view = _make_gather(a_view, b_view, row_pad, col_pad)
    out = _make_mlp(ga_view, gb_view, w2t_perm, b2r, e)
    return out
